# baseline (speedup 1.0000x reference)
import jax
import jax.numpy as jnp
from jax import lax
from jax.experimental import pallas as pl
from jax.experimental.pallas import tpu as pltpu
from jax.experimental.pallas import tpu_sc as plsc

VOC = 1000000
D = 64
B = 4096
T = 200

NUM_CORES = 2
NUM_SUBCORES = 16
NUM_WORKERS = NUM_CORES * NUM_SUBCORES
BLK = B // NUM_WORKERS

NBUF = 4
NGROUPS = T // NBUF

SCALE = 8.0
OPITCH = 129


def _position_embedding(max_len, d_model):
    angle = jnp.arange(d_model, dtype=jnp.float32)
    angle = 10000.0 ** (2.0 * (angle / d_model))
    angle = jnp.arange(max_len, dtype=jnp.float32)[:, None] / angle
    values = jnp.stack([jnp.sin(angle[:, 0::2]), jnp.cos(angle[:, 1::2])], axis=2)
    return values.reshape(max_len, -1).astype(jnp.float32)


def _gather(table_hbm, idx_all, gbuf, sem, t):
    return pltpu.make_async_copy(table_hbm.at[idx_all.at[t]], gbuf, sem)


def _out_copy(out_hbm, obuf, sem, t, wid):
    return [
        pltpu.make_async_copy(
            obuf.at[pl.ds(dt * 8, 8), pl.ds(0, 128)], out_hbm.at[t, dt, wid], sem
        )
        for dt in range(8)
    ]


def _compute(gbuf, obuf, pos_v, t):
    iota = lax.iota(jnp.int32, 16)
    d_v = [c * 16 + iota for c in range(D // 16)]
    pos = [pos_v[t, pl.ds(c * 16, 16)] for c in range(D // 16)]

    @plsc.parallel_loop(0, BLK, unroll=16)
    def per_row(r):
        bb = jnp.full((16,), r, jnp.int32)
        for c in range(D // 16):
            v = gbuf[r, pl.ds(c * 16, 16)] * SCALE + pos[c]
            plsc.store_scatter(obuf, [d_v[c], bb], v)


def _body(seq_hbm, table_hbm, pos_hbm, out_hbm, idx_all, pos_v,
          g0, g1, g2, g3, o0, o1, o2, o3,
          gs0, gs1, gs2, gs3, os0, os1, os2, os3):
    gbuf = (g0, g1, g2, g3)
    obuf = (o0, o1, o2, o3)
    gsem = (gs0, gs1, gs2, gs3)
    osem = (os0, os1, os2, os3)

    wid = lax.axis_index("s") * NUM_CORES + lax.axis_index("c")
    b0 = wid * BLK

    pltpu.sync_copy(seq_hbm.at[:, pl.ds(b0, BLK)], idx_all)
    pltpu.sync_copy(pos_hbm, pos_v)

    for b in range(NBUF):
        _gather(table_hbm, idx_all, gbuf[b], gsem[b], b).start()

    def chunk(t, b, prefetch, first_round):
        _gather(table_hbm, idx_all, gbuf[b], gsem[b], t).wait()
        if not first_round:
            for c in _out_copy(out_hbm, obuf[b], osem[b], t - NBUF, wid):
                c.wait()
        _compute(gbuf[b], obuf[b], pos_v, t)
        for c in _out_copy(out_hbm, obuf[b], osem[b], t, wid):
            c.start()
        if prefetch:
            _gather(table_hbm, idx_all, gbuf[b], gsem[b], t + NBUF).start()

    def group(g, carry):
        for b in range(NBUF):
            chunk(g * NBUF + b, b, True, False)
        return carry

    for b in range(NBUF):
        chunk(b, b, True, True)
    lax.fori_loop(1, NGROUPS - 1, group, 0)
    for b in range(NBUF):
        chunk((NGROUPS - 1) * NBUF + b, b, False, False)
    for b in range(NBUF):
        for c in _out_copy(out_hbm, obuf[b], osem[b], (NGROUPS - 1) * NBUF + b, wid):
            c.wait()


@jax.jit
def _run(seqT, table, pos):
    mesh = plsc.VectorSubcoreMesh(core_axis_name="c", subcore_axis_name="s")
    kern = pl.kernel(
        _body,
        out_type=jax.ShapeDtypeStruct((T, 8, NUM_WORKERS, 8, 128), jnp.float32),
        mesh=mesh,
        scratch_types=(
            [pltpu.VMEM((T, BLK), jnp.int32),
             pltpu.VMEM((T, D), jnp.float32)]
            + [pltpu.VMEM((BLK, D), jnp.float32) for _ in range(NBUF)]
            + [pltpu.VMEM((D, OPITCH), jnp.float32) for _ in range(NBUF)]
            + [pltpu.SemaphoreType.DMA for _ in range(2 * NBUF)]
        ),
        compiler_params=pltpu.CompilerParams(
            use_tc_tiling_on_sc=False,
            needs_layout_passes=False,
            disable_bounds_checks=True,
        ),
    )
    out5d = kern(seqT, table, pos)
    return out5d.transpose(2, 4, 0, 1, 3).reshape(B, T, D)


def kernel(sequences, table):
    pos = _position_embedding(T, D)
    return _run(jnp.swapaxes(sequences, 0, 1), table, pos)


# speedup vs baseline: 1.0507x; 1.0507x over previous
import jax
import jax.numpy as jnp
from jax import lax
from jax.experimental import pallas as pl
from jax.experimental.pallas import tpu as pltpu
from jax.experimental.pallas import tpu_sc as plsc

VOC = 1000000
D = 64
B = 4096
T = 200

NUM_CORES = 2
NUM_SUBCORES = 16
NUM_WORKERS = NUM_CORES * NUM_SUBCORES
BLK = B // NUM_WORKERS

NBUF = 4
NGROUPS = T // NBUF

SCALE = 8.0
OPITCH = 129


def _position_embedding(max_len, d_model):
    angle = jnp.arange(d_model, dtype=jnp.float32)
    angle = 10000.0 ** (2.0 * (angle / d_model))
    angle = jnp.arange(max_len, dtype=jnp.float32)[:, None] / angle
    values = jnp.stack([jnp.sin(angle[:, 0::2]), jnp.cos(angle[:, 1::2])], axis=2)
    return values.reshape(max_len, -1).astype(jnp.float32)


def _gather(table_hbm, idx_all, gbuf, sem, t):
    return pltpu.make_async_copy(table_hbm.at[idx_all.at[t]], gbuf, sem)


def _out_copy(out_hbm, obuf, sem, t, wid):
    return [
        pltpu.make_async_copy(
            obuf.at[pl.ds(dt * 8, 8), pl.ds(0, 128)], out_hbm.at[t, dt, wid], sem
        )
        for dt in range(8)
    ]


def _compute(gbuf, obuf, pos_v, t):
    iota = lax.iota(jnp.int32, 16)
    d_v = [c * 16 + iota for c in range(D // 16)]
    pos = [pos_v[t, pl.ds(c * 16, 16)] for c in range(D // 16)]

    @plsc.parallel_loop(0, BLK, unroll=8)
    def per_row(r):
        bb = jnp.full((16,), r, jnp.int32)
        for c in range(D // 16):
            v = gbuf[r, pl.ds(c * 16, 16)] * SCALE + pos[c]
            plsc.store_scatter(obuf, [d_v[c], bb], v)


def _body(seq_hbm, table_hbm, pos_hbm, out_hbm, idx_all, pos_v,
          g0, g1, g2, g3, o0, o1, o2, o3,
          gs0, gs1, gs2, gs3, os0, os1, os2, os3):
    gbuf = (g0, g1, g2, g3)
    obuf = (o0, o1, o2, o3)
    gsem = (gs0, gs1, gs2, gs3)
    osem = (os0, os1, os2, os3)

    wid = lax.axis_index("s") * NUM_CORES + lax.axis_index("c")
    b0 = wid * BLK

    pltpu.sync_copy(seq_hbm.at[:, pl.ds(b0, BLK)], idx_all)
    pltpu.sync_copy(pos_hbm, pos_v)

    for b in range(NBUF):
        _gather(table_hbm, idx_all, gbuf[b], gsem[b], b).start()

    def chunk(t, b, prefetch, first_round):
        _gather(table_hbm, idx_all, gbuf[b], gsem[b], t).wait()
        if not first_round:
            for c in _out_copy(out_hbm, obuf[b], osem[b], t - NBUF, wid):
                c.wait()
        _compute(gbuf[b], obuf[b], pos_v, t)
        for c in _out_copy(out_hbm, obuf[b], osem[b], t, wid):
            c.start()
        if prefetch:
            _gather(table_hbm, idx_all, gbuf[b], gsem[b], t + NBUF).start()

    def group(g, carry):
        for b in range(NBUF):
            chunk(g * NBUF + b, b, True, False)
        return carry

    for b in range(NBUF):
        chunk(b, b, True, True)
    lax.fori_loop(1, NGROUPS - 1, group, 0)
    for b in range(NBUF):
        chunk((NGROUPS - 1) * NBUF + b, b, False, False)
    for b in range(NBUF):
        for c in _out_copy(out_hbm, obuf[b], osem[b], (NGROUPS - 1) * NBUF + b, wid):
            c.wait()


@jax.jit
def _run(seqT, table, pos):
    mesh = plsc.VectorSubcoreMesh(core_axis_name="c", subcore_axis_name="s")
    kern = pl.kernel(
        _body,
        out_type=jax.ShapeDtypeStruct((T, 8, NUM_WORKERS, 8, 128), jnp.float32),
        mesh=mesh,
        scratch_types=(
            [pltpu.VMEM((T, BLK), jnp.int32),
             pltpu.VMEM((T, D), jnp.float32)]
            + [pltpu.VMEM((BLK, D), jnp.float32) for _ in range(NBUF)]
            + [pltpu.VMEM((D, OPITCH), jnp.float32) for _ in range(NBUF)]
            + [pltpu.SemaphoreType.DMA for _ in range(2 * NBUF)]
        ),
        compiler_params=pltpu.CompilerParams(
            use_tc_tiling_on_sc=False,
            needs_layout_passes=False,
            disable_bounds_checks=True,
        ),
    )
    out5d = kern(seqT, table, pos)
    return out5d.transpose(2, 4, 0, 1, 3).reshape(B, T, D)


def kernel(sequences, table):
    pos = _position_embedding(T, D)
    return _run(jnp.swapaxes(sequences, 0, 1), table, pos)
